# baseline (speedup 1.0000x reference)
import functools

import jax
import jax.numpy as jnp
from jax import lax
from jax.experimental import pallas as pl
from jax.experimental.pallas import tpu as pltpu
from jax.experimental.pallas import tpu_sc as plsc

DIM = 64
NUM_NEG = 5
NLOG = 1 + NUM_NEG
CHUNK = 128
LANES = 16


@functools.lru_cache(maxsize=None)
def _build_sc_kernel(B: int, NW: int):
    b_per_w = B // NW
    n_chunks = b_per_w // CHUNK
    mesh = plsc.VectorSubcoreMesh(core_axis_name="c", subcore_axis_name="s")

    @functools.partial(
        pl.kernel,
        mesh=mesh,
        compiler_params=pltpu.CompilerParams(
            use_tc_tiling_on_sc=False, needs_layout_passes=False
        ),
        out_type=jax.ShapeDtypeStruct((B * NLOG,), jnp.float32),
        scratch_types=[
            pltpu.VMEM((CHUNK,), jnp.int32),
            pltpu.VMEM((CHUNK,), jnp.int32),
            pltpu.VMEM((CHUNK * NUM_NEG,), jnp.int32),
            pltpu.VMEM((CHUNK, DIM), jnp.float32),
            pltpu.VMEM((CHUNK, DIM), jnp.float32),
            pltpu.VMEM((CHUNK * NUM_NEG, DIM), jnp.float32),
            pltpu.VMEM((CHUNK * NLOG,), jnp.float32),
            pltpu.SemaphoreType.DMA,
        ],
    )
    def sc_kernel(
        cen_hbm, pos_hbm, neg_hbm, table_hbm, out_hbm,
        idxc_v, idxp_v, idxn_v, rows_c, rows_p, rows_n, out_v, sem,
    ):
        wid = lax.axis_index("s") * 2 + lax.axis_index("c")
        lane = lax.iota(jnp.int32, LANES)

        for c in range(n_chunks):
            base = wid * b_per_w + c * CHUNK
            pltpu.sync_copy(cen_hbm.at[pl.ds(base, CHUNK)], idxc_v)
            pltpu.sync_copy(pos_hbm.at[pl.ds(base, CHUNK)], idxp_v)
            pltpu.sync_copy(
                neg_hbm.at[pl.ds(base * NUM_NEG, CHUNK * NUM_NEG)], idxn_v
            )
            cps = [
                pltpu.async_copy(table_hbm.at[idxc_v], rows_c, sem),
                pltpu.async_copy(table_hbm.at[idxp_v], rows_p, sem),
            ] + [
                pltpu.async_copy(
                    table_hbm.at[idxn_v.at[pl.ds(g * CHUNK, CHUNK)]],
                    rows_n.at[pl.ds(g * CHUNK, CHUNK)],
                    sem,
                )
                for g in range(NUM_NEG)
            ]
            for cp in cps:
                cp.wait()

            def group(g, _):
                bvec = g * LANES + lane
                acc = [jnp.zeros((LANES,), jnp.float32) for _ in range(NLOG)]
                for d in range(DIM):
                    dvec = jnp.full((LANES,), d, jnp.int32)
                    cen = plsc.load_gather(rows_c, [bvec, dvec])
                    acc[0] = acc[0] + cen * plsc.load_gather(
                        rows_p, [bvec, dvec]
                    )
                    for j in range(NUM_NEG):
                        acc[1 + j] = acc[1 + j] + cen * plsc.load_gather(
                            rows_n, [bvec * NUM_NEG + j, dvec]
                        )
                for j in range(NLOG):
                    prob = 1.0 / (1.0 + jnp.exp(-acc[j]))
                    plsc.store_scatter(out_v, [bvec * NLOG + j], prob)
                return 0

            lax.fori_loop(0, CHUNK // LANES, group, 0)

            pltpu.sync_copy(out_v, out_hbm.at[pl.ds(base * NLOG, CHUNK * NLOG)])

    return sc_kernel


def kernel(x_center, x_positive, x_negative, table):
    B = x_center.shape[0]
    NW = 32
    neg_flat = x_negative.reshape(B * NUM_NEG)
    flat = _build_sc_kernel(B, NW)(x_center, x_positive, neg_flat, table)
    return flat.reshape(B, NLOG)
